# Initial kernel scaffold; baseline (speedup 1.0000x reference)
import jax
import jax.numpy as jnp
from jax.experimental import pallas as pl


def kernel(fids, table):
    raise NotImplementedError("write your pallas kernel here")


# speedup vs baseline: 1.0692x; 1.0692x over previous
import functools

import jax
import jax.numpy as jnp
from jax import lax
from jax.experimental import pallas as pl
from jax.experimental.pallas import tpu as pltpu
from jax.experimental.pallas import tpu_sc as plsc

NW = 32
CHUNK = 128
INFLIGHT = 16
LANES = 16


def _lr_kernel(fids_hbm, table_hbm, out_hbm, idx_v, vals_v, out_v, sem):
    n_ch = idx_v.shape[0]
    rows_per_w = out_v.shape[0]
    f_per_row = (n_ch * CHUNK) // rows_per_w

    wid = lax.axis_index("s") * 2 + lax.axis_index("c")

    pltpu.sync_copy(fids_hbm.at[wid], idx_v)

    def gather_group(g, _):
        copies = []
        for b in range(INFLIGHT):
            j = g * INFLIGHT + b
            copies.append(
                pltpu.async_copy(
                    table_hbm.at[idx_v.at[j]],
                    vals_v.at[pl.ds(j * CHUNK, CHUNK)],
                    sem,
                )
            )
        for c in copies:
            c.wait()
        return 0

    lax.fori_loop(0, n_ch // INFLIGHT, gather_group, 0)

    def row_block(bb, _):
        base = bb * LANES

        def f_step(f, acc):
            return acc + vals_v[pl.ds(f * rows_per_w + base, LANES)]

        acc = lax.fori_loop(
            0, f_per_row, f_step, jnp.zeros((LANES,), jnp.float32)
        )
        out_v[pl.ds(base, LANES)] = acc
        return 0

    lax.fori_loop(0, rows_per_w // LANES, row_block, 0)

    pltpu.sync_copy(out_v, out_hbm.at[pl.ds(wid * rows_per_w, rows_per_w)])


def kernel(fids, table):
    B, F = fids.shape
    vocab = table.shape[0]
    rows_per_w = B // NW
    n_ch = (rows_per_w * F) // CHUNK
    assert rows_per_w * F == n_ch * CHUNK and n_ch % INFLIGHT == 0

    fids_r = (
        fids.reshape(NW, rows_per_w, F)
        .transpose(0, 2, 1)
        .reshape(NW, n_ch, CHUNK)
    )
    table_flat = table.reshape(vocab)

    mesh = plsc.VectorSubcoreMesh(core_axis_name="c", subcore_axis_name="s")
    run = functools.partial(
        pl.kernel,
        out_type=jax.ShapeDtypeStruct((B,), jnp.float32),
        mesh=mesh,
        scratch_types=[
            pltpu.VMEM((n_ch, CHUNK), jnp.int32),
            pltpu.VMEM((n_ch * CHUNK,), jnp.float32),
            pltpu.VMEM((rows_per_w,), jnp.float32),
            pltpu.SemaphoreType.DMA,
        ],
    )(_lr_kernel)
    return run(fids_r, table_flat)


# speedup vs baseline: 1.0867x; 1.0164x over previous
import functools

import jax
import jax.numpy as jnp
from jax import lax
from jax.experimental import pallas as pl
from jax.experimental.pallas import tpu as pltpu
from jax.experimental.pallas import tpu_sc as plsc

NW = 32
CHUNK = 128
INFLIGHT = 16
LANES = 16


def _lr_kernel(fids_hbm, table_hbm, out_hbm, idx_v, vals_v, out_v, sem):
    n_ch = idx_v.shape[0]
    rows_per_w = out_v.shape[0]
    f_per_row = (n_ch * CHUNK) // rows_per_w

    wid = lax.axis_index("s") * 2 + lax.axis_index("c")

    pltpu.sync_copy(fids_hbm.at[wid], idx_v)

    def fire(g):
        for b in range(INFLIGHT):
            j = g * INFLIGHT + b
            pltpu.async_copy(
                table_hbm.at[idx_v.at[j]],
                vals_v.at[pl.ds(j * CHUNK, CHUNK)],
                sem,
            )

    def drain(g):
        for b in range(INFLIGHT):
            j = g * INFLIGHT + b
            pltpu.make_async_copy(
                table_hbm.at[idx_v.at[j]],
                vals_v.at[pl.ds(j * CHUNK, CHUNK)],
                sem,
            ).wait()

    n_grp = n_ch // INFLIGHT
    fire(jnp.int32(0))

    def gather_group(g, _):
        fire(g)
        drain(g - 1)
        return 0

    lax.fori_loop(1, n_grp, gather_group, 0)
    drain(jnp.int32(n_grp - 1))

    def row_block(bb, _):
        base = bb * LANES

        def f_step(f, acc):
            return acc + vals_v[pl.ds(f * rows_per_w + base, LANES)]

        acc = lax.fori_loop(
            0, f_per_row, f_step, jnp.zeros((LANES,), jnp.float32)
        )
        out_v[pl.ds(base, LANES)] = acc
        return 0

    lax.fori_loop(0, rows_per_w // LANES, row_block, 0)

    pltpu.sync_copy(out_v, out_hbm.at[pl.ds(wid * rows_per_w, rows_per_w)])


def kernel(fids, table):
    B, F = fids.shape
    vocab = table.shape[0]
    rows_per_w = B // NW
    n_ch = (rows_per_w * F) // CHUNK
    assert rows_per_w * F == n_ch * CHUNK and n_ch % INFLIGHT == 0

    fids_r = (
        fids.reshape(NW, rows_per_w, F)
        .transpose(0, 2, 1)
        .reshape(NW, n_ch, CHUNK)
    )
    table_flat = table.reshape(vocab)

    mesh = plsc.VectorSubcoreMesh(core_axis_name="c", subcore_axis_name="s")
    run = functools.partial(
        pl.kernel,
        out_type=jax.ShapeDtypeStruct((B,), jnp.float32),
        mesh=mesh,
        scratch_types=[
            pltpu.VMEM((n_ch, CHUNK), jnp.int32),
            pltpu.VMEM((n_ch * CHUNK,), jnp.float32),
            pltpu.VMEM((rows_per_w,), jnp.float32),
            pltpu.SemaphoreType.DMA,
        ],
    )(_lr_kernel)
    return run(fids_r, table_flat)


# speedup vs baseline: 1.2165x; 1.1194x over previous
import functools

import jax
import jax.numpy as jnp
from jax import lax
from jax.experimental import pallas as pl
from jax.experimental.pallas import tpu as pltpu
from jax.experimental.pallas import tpu_sc as plsc

NW = 32
PADF = 128
VROW = 112
INFLIGHT = 16
LANES = 16
HALF = 256

_GDN = lax.GatherDimensionNumbers(
    offset_dims=(), collapsed_slice_dims=(0,), start_index_map=(0,)
)


def _permute(v, idx):
    return lax.gather(
        v, idx[:, None], _GDN, slice_sizes=(1,),
        mode=lax.GatherScatterMode.PROMISE_IN_BOUNDS,
    )


def _lr_kernel(fids_hbm, table_hbm, out_hbm, idx_v, vals_v, out_v, sem):
    rows_per_w = out_v.shape[0]
    f_per_row = 100

    wid = lax.axis_index("s") * 2 + lax.axis_index("c")
    base = wid * rows_per_w

    zeros16 = jnp.zeros((LANES,), jnp.float32)
    iota = lax.iota(jnp.int32, LANES)
    perms = [iota ^ s for s in (8, 4, 2, 1)]

    for h in range(rows_per_w // HALF):
        pltpu.sync_copy(
            fids_hbm.at[pl.ds(base + h * HALF, HALF), :], idx_v
        )

        def zero_tail(j, _):
            vals_v[j, pl.ds(96, LANES)] = zeros16
            return 0

        lax.fori_loop(0, HALF, zero_tail, 0)

        def fire(g):
            for b in range(INFLIGHT):
                j = g * INFLIGHT + b
                pltpu.async_copy(
                    table_hbm.at[idx_v.at[j, pl.ds(0, f_per_row)]],
                    vals_v.at[j, pl.ds(0, f_per_row)],
                    sem,
                )

        def drain(g):
            for b in range(INFLIGHT):
                j = g * INFLIGHT + b
                pltpu.make_async_copy(
                    table_hbm.at[idx_v.at[j, pl.ds(0, f_per_row)]],
                    vals_v.at[j, pl.ds(0, f_per_row)],
                    sem,
                ).wait()

        n_grp = HALF // INFLIGHT
        fire(jnp.int32(0))

        def gather_group(g, _):
            fire(g)
            drain(g - 1)
            return 0

        lax.fori_loop(1, n_grp, gather_group, 0)
        drain(jnp.int32(n_grp - 1))

        def row_block(j16, _):
            acc = zeros16
            for r in range(LANES):
                j = j16 * LANES + r
                s = vals_v[j, pl.ds(0, LANES)]
                for k in range(1, VROW // LANES):
                    s = s + vals_v[j, pl.ds(k * LANES, LANES)]
                for p in perms:
                    s = s + _permute(s, p)
                acc = jnp.where(iota == r, s, acc)
            out_v[pl.ds(h * HALF + j16 * LANES, LANES)] = acc
            return 0

        lax.fori_loop(0, HALF // LANES, row_block, 0)

    pltpu.sync_copy(out_v, out_hbm.at[pl.ds(base, rows_per_w)])


def kernel(fids, table):
    B, F = fids.shape
    vocab = table.shape[0]
    rows_per_w = B // NW
    assert B == NW * rows_per_w and F == 100 and rows_per_w % HALF == 0

    fids_p = jnp.pad(fids, ((0, 0), (0, PADF - F)))
    table_flat = table.reshape(vocab)

    mesh = plsc.VectorSubcoreMesh(core_axis_name="c", subcore_axis_name="s")
    run = functools.partial(
        pl.kernel,
        out_type=jax.ShapeDtypeStruct((B,), jnp.float32),
        mesh=mesh,
        scratch_types=[
            pltpu.VMEM((HALF, PADF), jnp.int32),
            pltpu.VMEM((HALF, VROW), jnp.float32),
            pltpu.VMEM((rows_per_w,), jnp.float32),
            pltpu.SemaphoreType.DMA,
        ],
    )(_lr_kernel)
    return run(fids_p, table_flat)


# speedup vs baseline: 1.3666x; 1.1233x over previous
import functools

import jax
import jax.numpy as jnp
from jax import lax
from jax.experimental import pallas as pl
from jax.experimental.pallas import tpu as pltpu
from jax.experimental.pallas import tpu_sc as plsc

NW = 32
PADF = 128
VROW = 104
GRP = 16
LANES = 16

_GDN = lax.GatherDimensionNumbers(
    offset_dims=(), collapsed_slice_dims=(0,), start_index_map=(0,)
)


def _permute(v, idx):
    return lax.gather(
        v, idx[:, None], _GDN, slice_sizes=(1,),
        mode=lax.GatherScatterMode.PROMISE_IN_BOUNDS,
    )


def _lr_kernel(fids_hbm, table_hbm, out_hbm, idx_v, vals_v, out_v, sem):
    rows_per_w = out_v.shape[0]
    f_per_row = 100

    wid = lax.axis_index("s") * 2 + lax.axis_index("c")
    base = wid * rows_per_w

    pltpu.sync_copy(fids_hbm.at[pl.ds(base, rows_per_w), :], idx_v)

    iota = lax.iota(jnp.int32, LANES)
    perms = [iota ^ s for s in (8, 4, 2, 1)]
    tailmask = (iota >= 8) & (iota < 12)
    zeros16 = jnp.zeros((LANES,), jnp.float32)

    def fire(g):
        for b in range(GRP):
            j = g * GRP + b
            pltpu.async_copy(
                table_hbm.at[idx_v.at[j, pl.ds(0, f_per_row)]],
                vals_v.at[pl.ds(j * VROW, f_per_row)],
                sem,
            )

    def drain(g):
        for b in range(GRP):
            j = g * GRP + b
            pltpu.make_async_copy(
                table_hbm.at[idx_v.at[j, pl.ds(0, f_per_row)]],
                vals_v.at[pl.ds(j * VROW, f_per_row)],
                sem,
            ).wait()

    def reduce_grp(g):
        acc = zeros16
        for r in range(GRP):
            j = g * GRP + r
            s = vals_v[pl.ds(j * VROW, LANES)]
            for k in range(1, 6):
                s = s + vals_v[pl.ds(j * VROW + k * LANES, LANES)]
            tail = vals_v[pl.ds(j * VROW + 88, LANES)]
            s = s + jnp.where(tailmask, tail, 0.0)
            for p in perms:
                s = s + _permute(s, p)
            acc = jnp.where(iota == r, s, acc)
        out_v[pl.ds(g * GRP, GRP)] = acc

    n_grp = rows_per_w // GRP
    fire(jnp.int32(0))
    fire(jnp.int32(1))

    def pipe(g, _):
        fire(g)
        drain(g - 2)
        reduce_grp(g - 2)
        return 0

    lax.fori_loop(2, n_grp, pipe, 0)
    for g in (n_grp - 2, n_grp - 1):
        drain(jnp.int32(g))
        reduce_grp(jnp.int32(g))

    pltpu.sync_copy(out_v, out_hbm.at[pl.ds(base, rows_per_w)])


def kernel(fids, table):
    B, F = fids.shape
    vocab = table.shape[0]
    rows_per_w = B // NW
    assert B == NW * rows_per_w and F == 100 and rows_per_w % GRP == 0

    fids_p = jnp.pad(fids, ((0, 0), (0, PADF - F)))
    table_flat = table.reshape(vocab)

    mesh = plsc.VectorSubcoreMesh(core_axis_name="c", subcore_axis_name="s")
    run = functools.partial(
        pl.kernel,
        out_type=jax.ShapeDtypeStruct((B,), jnp.float32),
        mesh=mesh,
        scratch_types=[
            pltpu.VMEM((rows_per_w, PADF), jnp.int32),
            pltpu.VMEM((rows_per_w * VROW,), jnp.float32),
            pltpu.VMEM((rows_per_w,), jnp.float32),
            pltpu.SemaphoreType.DMA,
        ],
    )(_lr_kernel)
    return run(fids_p, table_flat)
